# baseline (speedup 1.0000x reference)
import functools

import jax
import jax.numpy as jnp
from jax import lax
from jax.experimental import pallas as pl
from jax.experimental.pallas import tpu as pltpu
from jax.experimental.pallas import tpu_sc as plsc

B = 16384
N_IN = 256
SIZES = (128, 128, 128, 128, 64)
TOTALS = (256, 384, 512, 640, 768)
K = 16
BLK = 4096
OFF = (0, 128, 256, 384, 512, 576)

SZ_WX = 256 * 576
SZ_W0A = 128 * 128
SZ_W01 = 256 * 320
SZ_W2A = 128 * 128
SZ_W23 = 256 * 64
BASE_W0A = SZ_WX
BASE_W01 = BASE_W0A + SZ_W0A
BASE_W2A = BASE_W01 + SZ_W01
BASE_W23 = BASE_W2A + SZ_W2A
NTAB = BASE_W23 + SZ_W23
NW = 32
CHUNK = NTAB // NW
NE = sum(s * K for s in SIZES)
NEP = NE


def _edge_offsets(idxs, ws):
    offs, vals = [], []
    for li in range(5):
        sz = SIZES[li]
        r = idxs[li].astype(jnp.int32)
        r = jnp.where(r < N_IN, N_IN - 1 - r, r)
        n = jnp.arange(sz, dtype=jnp.int32)[:, None]
        o_wx = r * 576 + (OFF[li] + n)
        if li == 1:
            alt = BASE_W0A + (r - 256) * 128 + n
        elif li >= 2:
            alt = BASE_W01 + (r - 256) * 320 + (OFF[li] - 256 + n)
        else:
            alt = o_wx
        o = jnp.where(r < 256, o_wx, alt)
        if li == 3:
            o = jnp.where(r >= 512, BASE_W2A + (r - 512) * 128 + n, o)
        if li == 4:
            o = jnp.where(r >= 512, BASE_W23 + (r - 512) * 64 + n, o)
        offs.append(o.reshape(-1))
        vals.append(ws[li].reshape(-1))
    return jnp.concatenate(offs), jnp.concatenate(vals)


def _make_sc_densify():
    mesh = plsc.VectorSubcoreMesh(core_axis_name="c", subcore_axis_name="s")

    @functools.partial(
        pl.kernel, mesh=mesh,
        compiler_params=pltpu.CompilerParams(needs_layout_passes=False),
        out_type=jax.ShapeDtypeStruct((NTAB,), jnp.float32),
        scratch_types=[
            pltpu.VMEM((NEP,), jnp.int32),
            pltpu.VMEM((NEP,), jnp.float32),
            pltpu.VMEM((CHUNK,), jnp.float32),
        ],
    )
    def sc_densify(off_hbm, val_hbm, out_hbm, off_v, val_v, acc_v):
        wid = lax.axis_index("s") * 2 + lax.axis_index("c")
        base = wid * CHUNK
        pltpu.sync_copy(off_hbm, off_v)
        pltpu.sync_copy(val_hbm, val_v)

        zero = jnp.zeros((16,), jnp.float32)

        def zbody(i, carry):
            acc_v[pl.ds(i * 16, 16)] = zero
            return carry

        lax.fori_loop(0, CHUNK // 16, zbody, 0)

        def ebody(i, carry):
            off = off_v[pl.ds(i * 16, 16)]
            val = val_v[pl.ds(i * 16, 16)]
            loc = off - base
            msk = (loc >= 0) & (loc < CHUNK)
            loc = jnp.where(msk, loc, 0)
            plsc.addupdate_scatter(acc_v, [loc], val, mask=msk)
            return carry

        lax.fori_loop(0, NEP // 16, ebody, 0)
        pltpu.sync_copy(acc_v, out_hbm.at[pl.ds(base, CHUNK)])

    return sc_densify


def _body(wx, w0a, w01, w2a, w23, x_ref, out_ref):
    dot = functools.partial(jnp.dot, preferred_element_type=jnp.float32)
    x = x_ref[...]
    X = dot(x, wx[...])
    h0 = jnp.tanh(X[:, 0:128])
    h1 = jax.nn.relu(X[:, 128:256] + dot(h0, w0a[...]))
    T = dot(jnp.concatenate([h0, h1], axis=1), w01[...])
    h2 = jax.nn.sigmoid(X[:, 256:384] + T[:, 0:128])
    h3 = jnp.tanh(X[:, 384:512] + T[:, 128:256] + dot(h2, w2a[...]))
    out_ref[...] = (X[:, 512:576] + T[:, 256:320]
                    + dot(jnp.concatenate([h2, h3], axis=1), w23[...]))


def kernel(x, idx0, idx1, idx2, idx3, idx4, w0, w1, w2, w3, w4):
    idxs = (idx0, idx1, idx2, idx3, idx4)
    ws = (w0, w1, w2, w3, w4)
    offs, vals = _edge_offsets(idxs, ws)

    tab = _make_sc_densify()(offs, vals)
    wx = tab[:SZ_WX].reshape(256, 576)
    w0a = tab[BASE_W0A:BASE_W01].reshape(128, 128)
    w01 = tab[BASE_W01:BASE_W2A].reshape(256, 320)
    w2a = tab[BASE_W2A:BASE_W23].reshape(128, 128)
    w23 = tab[BASE_W23:].reshape(256, 64)

    grid = (B // BLK,)
    out = pl.pallas_call(
        _body,
        grid=grid,
        in_specs=[
            pl.BlockSpec((256, 576), lambda i: (0, 0)),
            pl.BlockSpec((128, 128), lambda i: (0, 0)),
            pl.BlockSpec((256, 320), lambda i: (0, 0)),
            pl.BlockSpec((128, 128), lambda i: (0, 0)),
            pl.BlockSpec((256, 64), lambda i: (0, 0)),
            pl.BlockSpec((BLK, N_IN), lambda i: (i, 0)),
        ],
        out_specs=pl.BlockSpec((BLK, SIZES[-1]), lambda i: (i, 0)),
        out_shape=jax.ShapeDtypeStruct((B, SIZES[-1]), jnp.float32),
    )(wx, w0a, w01, w2a, w23, x)
    return out


# speedup vs baseline: 2.3742x; 2.3742x over previous
import functools

import jax
import jax.numpy as jnp
from jax.experimental import pallas as pl
from jax.experimental.pallas import tpu as pltpu

B = 16384
N_IN = 256
SIZES = (128, 128, 128, 128, 64)
TOTALS = (256, 384, 512, 640, 768)
K = 16
BLK = 4096
OFF = (0, 128, 256, 384, 512, 576)


def _body(idx0, idx1, idx2, idx3, idx4, w0, w1, w2, w3, w4, x_ref,
          out_ref, wx, w0a, w01, w2a, w23):
    idx_refs = (idx0, idx1, idx2, idx3, idx4)
    w_refs = (w0, w1, w2, w3, w4)

    @pl.when(pl.program_id(0) == 0)
    def _densify():
        for li in range(5):
            sz = SIZES[li]
            rows = TOTALS[li]
            idx = idx_refs[li][...]
            idx = jnp.where(idx < N_IN, N_IN - 1 - idx, idx)
            w = w_refs[li][...]
            row_id = jax.lax.broadcasted_iota(jnp.int32, (rows, sz), 0)
            m = jnp.zeros((rows, sz), dtype=jnp.float32)
            for k in range(K):
                m = m + jnp.where(row_id == idx[k][None, :],
                                  w[k][None, :], 0.0)
            c0, c1 = OFF[li], OFF[li] + sz
            wx[:, c0:c1] = m[:256]
            if li == 1:
                w0a[...] = m[256:384]
            if li >= 2:
                w01[0:128, c0 - 256:c1 - 256] = m[256:384]
                w01[128:256, c0 - 256:c1 - 256] = (
                    m[384:512] if rows > 384
                    else jnp.zeros((128, sz), jnp.float32))
            if li == 3:
                w2a[...] = m[512:640]
            if li == 4:
                w23[0:128, :] = m[512:640]
                w23[128:256, :] = m[640:768]

    dot = functools.partial(jnp.dot, preferred_element_type=jnp.float32)
    x = x_ref[...]
    X = dot(x, wx[...])
    h0 = jnp.tanh(X[:, 0:128])
    h1 = jax.nn.relu(X[:, 128:256] + dot(h0, w0a[...]))
    T = dot(jnp.concatenate([h0, h1], axis=1), w01[...])
    h2 = jax.nn.sigmoid(X[:, 256:384] + T[:, 0:128])
    h3 = jnp.tanh(X[:, 384:512] + T[:, 128:256] + dot(h2, w2a[...]))
    out_ref[...] = (X[:, 512:576] + T[:, 256:320]
                    + dot(jnp.concatenate([h2, h3], axis=1), w23[...]))


def kernel(x, idx0, idx1, idx2, idx3, idx4, w0, w1, w2, w3, w4):
    idxs = [a.T for a in (idx0, idx1, idx2, idx3, idx4)]
    ws = [a.T for a in (w0, w1, w2, w3, w4)]

    grid = (B // BLK,)
    out = pl.pallas_call(
        _body,
        grid=grid,
        in_specs=[pl.BlockSpec((K, SIZES[li]), lambda i: (0, 0))
                  for li in range(5)] * 2
        + [pl.BlockSpec((BLK, N_IN), lambda i: (i, 0))],
        out_specs=pl.BlockSpec((BLK, SIZES[-1]), lambda i: (i, 0)),
        out_shape=jax.ShapeDtypeStruct((B, SIZES[-1]), jnp.float32),
        scratch_shapes=[
            pltpu.VMEM((256, 576), jnp.float32),
            pltpu.VMEM((128, 128), jnp.float32),
            pltpu.VMEM((256, 320), jnp.float32),
            pltpu.VMEM((128, 128), jnp.float32),
            pltpu.VMEM((256, 64), jnp.float32),
        ],
    )(*idxs, *ws, x)
    return out
